# baseline (speedup 1.0000x reference)
import functools

import numpy as np
import jax
import jax.numpy as jnp
from jax import lax
from jax.experimental import pallas as pl
from jax.experimental.pallas import tpu as pltpu
from jax.experimental.pallas import tpu_sc as plsc

NUM_EMB = 1_000_000
D = 64
N_SEQ = 4096
S_LEN = 200
B = N_SEQ * S_LEN
NW = 32
ROWS_PER_W = B // NW
CHUNK = 320
CHUNKS_PER_W = ROWS_PER_W // CHUNK
LANES = 16
NBUF = 4


def _pe_table() -> np.ndarray:
    position = np.arange(S_LEN)[:, None].astype(np.float32)
    div_term = np.exp(np.arange(0, D, 2).astype(np.float32) * (-np.log(10000.0) / D))
    pe = np.zeros((S_LEN, D), dtype=np.float32)
    pe[:, 0::2] = np.sin(position * div_term)
    pe[:, 1::2] = np.cos(position * div_term)
    return pe


_PE = _pe_table()

_mesh = plsc.VectorSubcoreMesh(core_axis_name="c", subcore_axis_name="s")


@functools.partial(
    pl.kernel,
    out_type=jax.ShapeDtypeStruct((B, D), jnp.float32),
    mesh=_mesh,
    scratch_types=[
        pltpu.VMEM((ROWS_PER_W,), jnp.int32),
        pltpu.VMEM((S_LEN, D), jnp.float32),
    ]
    + [pltpu.VMEM((CHUNK, D), jnp.float32)] * NBUF
    + [pltpu.SemaphoreType.DMA] * (2 * NBUF),
    compiler_params=pltpu.CompilerParams(use_tc_tiling_on_sc=False),
)
def _embed_sc(table_hbm, idx_hbm, pe_hbm, out_hbm, idx_v, pe_v, *bufs_and_sems):
    bufs = bufs_and_sems[:NBUF]
    gsems = bufs_and_sems[NBUF:2 * NBUF]
    ssems = bufs_and_sems[2 * NBUF:]
    wid = lax.axis_index("s") * 2 + lax.axis_index("c")
    base = wid * ROWS_PER_W
    pltpu.sync_copy(idx_hbm.at[wid], idx_v)
    pltpu.sync_copy(pe_hbm, pe_v)

    def gather(ci, k):
        off = pl.multiple_of(ci * CHUNK, 8)
        return pltpu.make_async_copy(
            table_hbm.at[idx_v.at[pl.ds(off, CHUNK)]], bufs[k], gsems[k])

    def store(ci, k):
        return pltpu.make_async_copy(
            bufs[k], out_hbm.at[pl.ds(base + ci * CHUNK, CHUNK)], ssems[k])

    def add_pe(ci, k):
        buf = bufs[k]

        def row_body(r, c2):
            s = lax.rem(ci * CHUNK + r, S_LEN)
            for j in range(D // LANES):
                sl = pl.ds(j * LANES, LANES)
                plsc.addupdate(buf.at[r, sl], pe_v[s, sl])
            return c2

        lax.fori_loop(0, CHUNK, row_body, 0, unroll=4)

    gather(0, 0).start()
    gather(1, 1).start()

    def group_body(g, carry):
        for k in range(NBUF):
            ci = g * NBUF + k
            kn = (k + 2) % NBUF

            @pl.when(jnp.logical_and(ci >= 2, ci <= CHUNKS_PER_W - 3))
            def _():
                store(ci - 2, kn).wait()

            @pl.when(ci <= CHUNKS_PER_W - 3)
            def _():
                gather(ci + 2, kn).start()

            gather(ci, k).wait()
            add_pe(ci, k)
            store(ci, k).start()
        return carry

    lax.fori_loop(0, CHUNKS_PER_W // NBUF, group_body, 0)
    for k in range(NBUF):
        store(CHUNKS_PER_W - NBUF + k, k).wait()


def kernel(pre_embedding, preembed_mask, embed_table):
    idx = pre_embedding.astype(jnp.int32).reshape(NW, ROWS_PER_W)
    pe = jnp.asarray(_PE)
    out = _embed_sc(embed_table, idx, pe)
    return out.reshape(N_SEQ, S_LEN, D), preembed_mask


# speedup vs baseline: 1.0963x; 1.0963x over previous
import functools

import numpy as np
import jax
import jax.numpy as jnp
from jax import lax
from jax.experimental import pallas as pl
from jax.experimental.pallas import tpu as pltpu
from jax.experimental.pallas import tpu_sc as plsc

NUM_EMB = 1_000_000
D = 64
N_SEQ = 4096
S_LEN = 200
B = N_SEQ * S_LEN
NW = 32
ROWS_PER_W = B // NW
CHUNK = 256
CHUNKS_PER_W = ROWS_PER_W // CHUNK
LANES = 16
TPAD = CHUNK + 8


def _pe_table() -> np.ndarray:
    position = np.arange(S_LEN)[:, None].astype(np.float32)
    div_term = np.exp(np.arange(0, D, 2).astype(np.float32) * (-np.log(10000.0) / D))
    pe = np.zeros((S_LEN, D), dtype=np.float32)
    pe[:, 0::2] = np.sin(position * div_term)
    pe[:, 1::2] = np.cos(position * div_term)
    return pe


_PE = _pe_table()

_mesh = plsc.VectorSubcoreMesh(core_axis_name="c", subcore_axis_name="s")


@functools.partial(
    pl.kernel,
    out_type=jax.ShapeDtypeStruct((S_LEN, D, N_SEQ), jnp.float32),
    mesh=_mesh,
    scratch_types=[
        pltpu.VMEM((ROWS_PER_W,), jnp.int32),
        pltpu.VMEM((S_LEN, D), jnp.float32),
        pltpu.VMEM((CHUNK, D), jnp.float32),
        pltpu.VMEM((CHUNK, D), jnp.float32),
        pltpu.VMEM((D, TPAD), jnp.float32),
        pltpu.VMEM((D, TPAD), jnp.float32),
        pltpu.SemaphoreType.DMA,
        pltpu.SemaphoreType.DMA,
        pltpu.SemaphoreType.DMA,
        pltpu.SemaphoreType.DMA,
    ],
    compiler_params=pltpu.CompilerParams(
        use_tc_tiling_on_sc=False, needs_layout_passes=False),
)
def _embed_sc(table_hbm, idx_hbm, pe_hbm, out_hbm, idx_v, pe_v,
              buf0, buf1, tb0, tb1, gs0, gs1, ss0, ss1):
    bufs = (buf0, buf1)
    tbs = (tb0, tb1)
    gsems = (gs0, gs1)
    ssems = (ss0, ss1)
    wid = lax.axis_index("s") * 2 + lax.axis_index("c")
    base = wid * ROWS_PER_W
    pltpu.sync_copy(idx_hbm.at[wid], idx_v)
    pltpu.sync_copy(pe_hbm, pe_v)

    iota = lax.iota(jnp.int32, LANES)
    dvec = [iota + j * LANES for j in range(D // LANES)]

    def gather(ci, k):
        off = pl.multiple_of(ci * CHUNK, 8)
        return pltpu.make_async_copy(
            table_hbm.at[idx_v.at[pl.ds(off, CHUNK)]], bufs[k], gsems[k])

    def store(ci, k):
        t0 = base + ci * CHUNK
        s = t0 // N_SEQ
        n0 = pl.multiple_of(lax.rem(t0, N_SEQ), CHUNK)
        return pltpu.make_async_copy(
            tbs[k].at[:, pl.ds(0, CHUNK)],
            out_hbm.at[s, :, pl.ds(n0, CHUNK)], ssems[k])

    def transpose_add(ci, k):
        buf, tb = bufs[k], tbs[k]
        s = (base + ci * CHUNK) // N_SEQ
        pe_regs = [pe_v[s, pl.ds(j * LANES, LANES)] for j in range(D // LANES)]

        def row_body(r, c2):
            rvec = jnp.broadcast_to(r, (LANES,)).astype(jnp.int32)
            for j in range(D // LANES):
                v = buf[r, pl.ds(j * LANES, LANES)] + pe_regs[j]
                plsc.store_scatter(tb, [dvec[j], rvec], v)
            return c2

        lax.fori_loop(0, CHUNK, row_body, 0, unroll=4)

    gather(0, 0).start()
    gather(1, 1).start()

    def group_body(g, carry):
        for k in range(2):
            ci = 2 * g + k
            gather(ci, k).wait()

            @pl.when(ci >= 2)
            def _():
                store(ci - 2, k).wait()

            transpose_add(ci, k)
            store(ci, k).start()

            @pl.when(ci <= CHUNKS_PER_W - 3)
            def _():
                gather(ci + 2, k).start()
        return carry

    lax.fori_loop(0, CHUNKS_PER_W // 2, group_body, 0)
    store(CHUNKS_PER_W - 2, 0).wait()
    store(CHUNKS_PER_W - 1, 1).wait()


def kernel(pre_embedding, preembed_mask, embed_table):
    idx_t = pre_embedding.astype(jnp.int32).T.reshape(NW, ROWS_PER_W)
    pe = jnp.asarray(_PE)
    out_t = _embed_sc(embed_table, idx_t, pe)
    return jnp.transpose(out_t, (2, 0, 1)), preembed_mask


# speedup vs baseline: 1.0979x; 1.0014x over previous
import functools

import numpy as np
import jax
import jax.numpy as jnp
from jax import lax
from jax.experimental import pallas as pl
from jax.experimental.pallas import tpu as pltpu
from jax.experimental.pallas import tpu_sc as plsc

NUM_EMB = 1_000_000
D = 64
N_SEQ = 4096
S_LEN = 200
B = N_SEQ * S_LEN
NW = 32
ROWS_PER_W = B // NW
CHUNK = 256
CHUNKS_PER_W = ROWS_PER_W // CHUNK
LANES = 16
TPAD = CHUNK + 1


def _pe_table() -> np.ndarray:
    position = np.arange(S_LEN)[:, None].astype(np.float32)
    div_term = np.exp(np.arange(0, D, 2).astype(np.float32) * (-np.log(10000.0) / D))
    pe = np.zeros((S_LEN, D), dtype=np.float32)
    pe[:, 0::2] = np.sin(position * div_term)
    pe[:, 1::2] = np.cos(position * div_term)
    return pe


_PE = _pe_table()

_mesh = plsc.VectorSubcoreMesh(core_axis_name="c", subcore_axis_name="s")


@functools.partial(
    pl.kernel,
    out_type=jax.ShapeDtypeStruct((S_LEN, D, N_SEQ), jnp.float32),
    mesh=_mesh,
    scratch_types=[
        pltpu.VMEM((ROWS_PER_W,), jnp.int32),
        pltpu.VMEM((S_LEN, D), jnp.float32),
        pltpu.VMEM((CHUNK, D), jnp.float32),
        pltpu.VMEM((CHUNK, D), jnp.float32),
        pltpu.VMEM((D, TPAD), jnp.float32),
        pltpu.VMEM((D, TPAD), jnp.float32),
        pltpu.SemaphoreType.DMA,
        pltpu.SemaphoreType.DMA,
        pltpu.SemaphoreType.DMA,
        pltpu.SemaphoreType.DMA,
    ],
    compiler_params=pltpu.CompilerParams(
        use_tc_tiling_on_sc=False, needs_layout_passes=False),
)
def _embed_sc(table_hbm, idx_hbm, pe_hbm, out_hbm, idx_v, pe_v,
              buf0, buf1, tb0, tb1, gs0, gs1, ss0, ss1):
    bufs = (buf0, buf1)
    tbs = (tb0, tb1)
    gsems = (gs0, gs1)
    ssems = (ss0, ss1)
    wid = lax.axis_index("s") * 2 + lax.axis_index("c")
    base = wid * ROWS_PER_W
    pltpu.sync_copy(idx_hbm.at[wid], idx_v)
    pltpu.sync_copy(pe_hbm, pe_v)

    iota = lax.iota(jnp.int32, LANES)
    dvec = [iota + j * LANES for j in range(D // LANES)]

    def gather(ci, k):
        off = pl.multiple_of(ci * CHUNK, 8)
        return pltpu.make_async_copy(
            table_hbm.at[idx_v.at[pl.ds(off, CHUNK)]], bufs[k], gsems[k])

    def store(ci, k):
        t0 = base + ci * CHUNK
        s = t0 // N_SEQ
        n0 = pl.multiple_of(lax.rem(t0, N_SEQ), CHUNK)
        return pltpu.make_async_copy(
            tbs[k].at[:, pl.ds(0, CHUNK)],
            out_hbm.at[s, :, pl.ds(n0, CHUNK)], ssems[k])

    def transpose_add(ci, k):
        buf, tb = bufs[k], tbs[k]
        s = (base + ci * CHUNK) // N_SEQ
        pe_regs = [pe_v[s, pl.ds(j * LANES, LANES)] for j in range(D // LANES)]

        def row_body(r, c2):
            rvec = jnp.broadcast_to(r, (LANES,)).astype(jnp.int32)
            for j in range(D // LANES):
                v = buf[r, pl.ds(j * LANES, LANES)] + pe_regs[j]
                plsc.store_scatter(tb, [dvec[j], rvec], v)
            return c2

        lax.fori_loop(0, CHUNK, row_body, 0, unroll=4)

    gather(0, 0).start()
    gather(1, 1).start()

    def group_body(g, carry):
        for k in range(2):
            ci = 2 * g + k
            gather(ci, k).wait()

            @pl.when(ci >= 2)
            def _():
                store(ci - 2, k).wait()

            transpose_add(ci, k)
            store(ci, k).start()

            @pl.when(ci <= CHUNKS_PER_W - 3)
            def _():
                gather(ci + 2, k).start()
        return carry

    lax.fori_loop(0, CHUNKS_PER_W // 2, group_body, 0)
    store(CHUNKS_PER_W - 2, 0).wait()
    store(CHUNKS_PER_W - 1, 1).wait()


def kernel(pre_embedding, preembed_mask, embed_table):
    idx_t = pre_embedding.astype(jnp.int32).T.reshape(NW, ROWS_PER_W)
    pe = jnp.asarray(_PE)
    out_t = _embed_sc(embed_table, idx_t, pe)
    return jnp.transpose(out_t, (2, 0, 1)), preembed_mask


# speedup vs baseline: 1.0982x; 1.0003x over previous
import functools

import numpy as np
import jax
import jax.numpy as jnp
from jax import lax
from jax.experimental import pallas as pl
from jax.experimental.pallas import tpu as pltpu
from jax.experimental.pallas import tpu_sc as plsc

NUM_EMB = 1_000_000
D = 64
N_SEQ = 4096
S_LEN = 200
B = N_SEQ * S_LEN
NW = 32
ROWS_PER_W = B // NW
CHUNK = 256
CHUNKS_PER_W = ROWS_PER_W // CHUNK
LANES = 16
TPAD = CHUNK + 1


def _pe_table() -> np.ndarray:
    position = np.arange(S_LEN)[:, None].astype(np.float32)
    div_term = np.exp(np.arange(0, D, 2).astype(np.float32) * (-np.log(10000.0) / D))
    pe = np.zeros((S_LEN, D), dtype=np.float32)
    pe[:, 0::2] = np.sin(position * div_term)
    pe[:, 1::2] = np.cos(position * div_term)
    return pe


_PE = _pe_table()

_mesh = plsc.VectorSubcoreMesh(core_axis_name="c", subcore_axis_name="s")


@functools.partial(
    pl.kernel,
    out_type=jax.ShapeDtypeStruct((S_LEN, D, N_SEQ), jnp.float32),
    mesh=_mesh,
    scratch_types=[
        pltpu.VMEM((ROWS_PER_W,), jnp.int32),
        pltpu.VMEM((S_LEN, D), jnp.float32),
        pltpu.VMEM((CHUNK, D), jnp.float32),
        pltpu.VMEM((CHUNK, D), jnp.float32),
        pltpu.VMEM((D, TPAD), jnp.float32),
        pltpu.VMEM((D, TPAD), jnp.float32),
        pltpu.SemaphoreType.DMA,
        pltpu.SemaphoreType.DMA,
        pltpu.SemaphoreType.DMA,
        pltpu.SemaphoreType.DMA,
    ],
    compiler_params=pltpu.CompilerParams(
        use_tc_tiling_on_sc=False, needs_layout_passes=False,
        disable_bounds_checks=True),
)
def _embed_sc(table_hbm, idx_hbm, pe_hbm, out_hbm, idx_v, pe_v,
              buf0, buf1, tb0, tb1, gs0, gs1, ss0, ss1):
    bufs = (buf0, buf1)
    tbs = (tb0, tb1)
    gsems = (gs0, gs1)
    ssems = (ss0, ss1)
    wid = lax.axis_index("s") * 2 + lax.axis_index("c")
    base = wid * ROWS_PER_W
    pltpu.sync_copy(idx_hbm.at[wid], idx_v)
    pltpu.sync_copy(pe_hbm, pe_v)

    iota = lax.iota(jnp.int32, LANES)
    dvec = [iota + j * LANES for j in range(D // LANES)]

    def gather(ci, k):
        off = pl.multiple_of(ci * CHUNK, 8)
        return pltpu.make_async_copy(
            table_hbm.at[idx_v.at[pl.ds(off, CHUNK)]], bufs[k], gsems[k])

    def store(ci, k):
        t0 = base + ci * CHUNK
        s = t0 // N_SEQ
        n0 = pl.multiple_of(lax.rem(t0, N_SEQ), CHUNK)
        return pltpu.make_async_copy(
            tbs[k].at[:, pl.ds(0, CHUNK)],
            out_hbm.at[s, :, pl.ds(n0, CHUNK)], ssems[k])

    def transpose_add(ci, k):
        buf, tb = bufs[k], tbs[k]
        s = (base + ci * CHUNK) // N_SEQ
        pe_regs = [pe_v[s, pl.ds(j * LANES, LANES)] for j in range(D // LANES)]

        def row_body(r, c2):
            rvec = jnp.broadcast_to(r, (LANES,))
            for j in range(D // LANES):
                v = buf[r, pl.ds(j * LANES, LANES)] + pe_regs[j]
                plsc.store_scatter(tb, [dvec[j], rvec], v)
            return c2

        lax.fori_loop(0, CHUNK, row_body, 0, unroll=4)

    gather(0, 0).start()
    gather(1, 1).start()

    def group_body(g, carry):
        for k in range(2):
            ci = 2 * g + k
            gather(ci, k).wait()

            @pl.when(ci >= 2)
            def _():
                store(ci - 2, k).wait()

            transpose_add(ci, k)
            store(ci, k).start()

            @pl.when(ci <= CHUNKS_PER_W - 3)
            def _():
                gather(ci + 2, k).start()
        return carry

    lax.fori_loop(0, CHUNKS_PER_W // 2, group_body, 0)
    store(CHUNKS_PER_W - 2, 0).wait()
    store(CHUNKS_PER_W - 1, 1).wait()


def kernel(pre_embedding, preembed_mask, embed_table):
    idx_t = pre_embedding.astype(jnp.int32).T.reshape(NW, ROWS_PER_W)
    pe = jnp.asarray(_PE)
    out_t = _embed_sc(embed_table, idx_t, pe)
    return jnp.transpose(out_t, (2, 0, 1)), preembed_mask


# speedup vs baseline: 1.5040x; 1.3695x over previous
import functools

import numpy as np
import jax
import jax.numpy as jnp
from jax import lax
from jax.experimental import pallas as pl
from jax.experimental.pallas import tpu as pltpu
from jax.experimental.pallas import tpu_sc as plsc

NUM_EMB = 1_000_000
D = 64
N_SEQ = 4096
S_LEN = 200
B = N_SEQ * S_LEN
NW = 32
ROWS_PER_W = B // NW
CHUNK = 256
CHUNKS_PER_W = ROWS_PER_W // CHUNK
LANES = 16
TPAD = CHUNK + 1


def _pe_table() -> np.ndarray:
    position = np.arange(S_LEN)[:, None].astype(np.float32)
    div_term = np.exp(np.arange(0, D, 2).astype(np.float32) * (-np.log(10000.0) / D))
    pe = np.zeros((S_LEN, D), dtype=np.float32)
    pe[:, 0::2] = np.sin(position * div_term)
    pe[:, 1::2] = np.cos(position * div_term)
    return pe


_PE = _pe_table()

_mesh = plsc.VectorSubcoreMesh(core_axis_name="c", subcore_axis_name="s")


@functools.partial(
    pl.kernel,
    out_type=jax.ShapeDtypeStruct((S_LEN, D, N_SEQ), jnp.float32),
    mesh=_mesh,
    scratch_types=[
        pltpu.VMEM((ROWS_PER_W,), jnp.int32),
        pltpu.VMEM((S_LEN, D), jnp.float32),
        pltpu.VMEM((CHUNK, D), jnp.float32),
        pltpu.VMEM((CHUNK, D), jnp.float32),
        pltpu.VMEM((D, TPAD), jnp.float32),
        pltpu.VMEM((D, TPAD), jnp.float32),
        pltpu.SemaphoreType.DMA,
        pltpu.SemaphoreType.DMA,
        pltpu.SemaphoreType.DMA,
        pltpu.SemaphoreType.DMA,
    ],
    compiler_params=pltpu.CompilerParams(
        use_tc_tiling_on_sc=False, needs_layout_passes=False,
        disable_bounds_checks=True),
)
def _embed_sc(table_hbm, idx_hbm, pe_hbm, out_hbm, idx_v, pe_v,
              buf0, buf1, tb0, tb1, gs0, gs1, ss0, ss1):
    bufs = (buf0, buf1)
    tbs = (tb0, tb1)
    gsems = (gs0, gs1)
    ssems = (ss0, ss1)
    wid = lax.axis_index("s") * 2 + lax.axis_index("c")
    base = wid * ROWS_PER_W
    pltpu.sync_copy(idx_hbm.at[wid], idx_v)
    pltpu.sync_copy(pe_hbm, pe_v)

    iota = lax.iota(jnp.int32, LANES)
    dvec = [iota + j * LANES for j in range(D // LANES)]

    def gather(ci, k):
        off = pl.multiple_of(ci * CHUNK, 8)
        return pltpu.make_async_copy(
            table_hbm.at[idx_v.at[pl.ds(off, CHUNK)]], bufs[k], gsems[k])

    def store(ci, k):
        t0 = base + ci * CHUNK
        s = t0 // N_SEQ
        n0 = pl.multiple_of(lax.rem(t0, N_SEQ), CHUNK)
        return pltpu.make_async_copy(
            tbs[k].at[:, pl.ds(0, CHUNK)],
            out_hbm.at[s, :, pl.ds(n0, CHUNK)], ssems[k])

    def transpose_add(ci, k):
        buf, tb = bufs[k], tbs[k]
        s = (base + ci * CHUNK) // N_SEQ
        pe_regs = [pe_v[s, pl.ds(j * LANES, LANES)] for j in range(D // LANES)]

        @plsc.parallel_loop(0, CHUNK, unroll=4)
        def row_body(r):
            rvec = jnp.broadcast_to(r, (LANES,))
            for j in range(D // LANES):
                v = buf[r, pl.ds(j * LANES, LANES)] + pe_regs[j]
                plsc.store_scatter(tb, [dvec[j], rvec], v)

    gather(0, 0).start()
    gather(1, 1).start()

    def group_body(g, carry):
        for k in range(2):
            ci = 2 * g + k
            gather(ci, k).wait()

            @pl.when(ci >= 2)
            def _():
                store(ci - 2, k).wait()

            transpose_add(ci, k)
            store(ci, k).start()

            @pl.when(ci <= CHUNKS_PER_W - 3)
            def _():
                gather(ci + 2, k).start()
        return carry

    lax.fori_loop(0, CHUNKS_PER_W // 2, group_body, 0)
    store(CHUNKS_PER_W - 2, 0).wait()
    store(CHUNKS_PER_W - 1, 1).wait()


def kernel(pre_embedding, preembed_mask, embed_table):
    idx_t = pre_embedding.astype(jnp.int32).T.reshape(NW, ROWS_PER_W)
    pe = jnp.asarray(_PE)
    out_t = _embed_sc(embed_table, idx_t, pe)
    return jnp.transpose(out_t, (2, 0, 1)), preembed_mask
